# baseline (speedup 1.0000x reference)
import functools
import math

import jax
import jax.numpy as jnp
from jax import lax
from jax.experimental import pallas as pl
from jax.experimental.pallas import tpu as pltpu
from jax.experimental.pallas import tpu_sc as plsc

_VOCAB = 1000000
_EMB = 64
_S = 200
_B = 4096
_N = _B * _S
_CHUNK = 128
_NC = 2
_NS = 16
_NW = _NC * _NS
_PER_W = _N // _NW
_CHUNKS_PER_W = _PER_W // _CHUNK
_SCALE = math.sqrt(_EMB)


def _make_kernel():
    mesh = plsc.VectorSubcoreMesh(core_axis_name="c", subcore_axis_name="s")

    @functools.partial(
        pl.kernel,
        mesh=mesh,
        out_type=jax.ShapeDtypeStruct((_N, _EMB), jnp.float32),
        compiler_params=pltpu.CompilerParams(use_tc_tiling_on_sc=False),
        scratch_types=[
            pltpu.VMEM((_CHUNKS_PER_W, _CHUNK), jnp.int32),
            pltpu.VMEM((2 * _S, _EMB), jnp.float32),
            pltpu.VMEM((_CHUNK, _EMB), jnp.float32),
            pltpu.SemaphoreType.DMA,
        ],
    )
    def k(x_hbm, tok_hbm, pos_hbm, out_hbm, idx_v, pos_v, buf, gsem):
        wid = lax.axis_index("s") * _NC + lax.axis_index("c")
        pltpu.sync_copy(x_hbm.at[pl.ds(wid * _CHUNKS_PER_W, _CHUNKS_PER_W)],
                        idx_v)
        pltpu.sync_copy(pos_hbm.at[pl.ds(0, _S)], pos_v.at[pl.ds(0, _S)])
        pltpu.sync_copy(pos_hbm.at[pl.ds(0, _S)], pos_v.at[pl.ds(_S, _S)])

        def chunk_body(j, carry):
            pltpu.async_copy(tok_hbm.at[idx_v.at[j]], buf, gsem).wait()
            base = lax.rem(j * _CHUNK, _S)

            def row_body(i, c2):
                s = base + i
                for d in range(_EMB // 16):
                    sl = pl.ds(d * 16, 16)
                    buf[i, sl] = buf[i, sl] * _SCALE + pos_v[s, sl]
                return c2

            lax.fori_loop(0, _CHUNK, row_body, 0)
            pltpu.sync_copy(
                buf,
                out_hbm.at[pl.ds((wid * _CHUNKS_PER_W + j) * _CHUNK, _CHUNK)])
            return carry

        lax.fori_loop(0, _CHUNKS_PER_W, chunk_body, 0)

    return k


_kernel_call = _make_kernel()


def kernel(x, token_table, pos_table):
    xf = x.reshape(_N // _CHUNK, _CHUNK).astype(jnp.int32)
    out = _kernel_call(xf, token_table, pos_table)
    return out.reshape(_B, _S, _EMB)


# speedup vs baseline: 1.1751x; 1.1751x over previous
import functools
import math

import jax
import jax.numpy as jnp
from jax import lax
from jax.experimental import pallas as pl
from jax.experimental.pallas import tpu as pltpu
from jax.experimental.pallas import tpu_sc as plsc

_VOCAB = 1000000
_EMB = 64
_S = 200
_B = 4096
_N = _B * _S
_CHUNK = 128
_NC = 2
_NS = 16
_NW = _NC * _NS
_PER_W = _N // _NW
_CHUNKS_PER_W = _PER_W // _CHUNK
_SCALE = math.sqrt(_EMB)


def _make_kernel():
    mesh = plsc.VectorSubcoreMesh(core_axis_name="c", subcore_axis_name="s")

    @functools.partial(
        pl.kernel,
        mesh=mesh,
        out_type=jax.ShapeDtypeStruct((_N, _EMB), jnp.float32),
        compiler_params=pltpu.CompilerParams(use_tc_tiling_on_sc=False),
        scratch_types=[
            pltpu.VMEM((_CHUNKS_PER_W, _CHUNK), jnp.int32),
            pltpu.VMEM((2 * _S, _EMB), jnp.float32),
            pltpu.VMEM((2, _CHUNK, _EMB), jnp.float32),
            pltpu.VMEM((2, _CHUNK, _EMB), jnp.float32),
            pltpu.SemaphoreType.DMA((2,)),
            pltpu.SemaphoreType.DMA((2,)),
        ],
    )
    def k(x_hbm, tok_hbm, pos_hbm, out_hbm, idx_v, pos_v, inb, outb, gsem,
          ssem):
        wid = lax.axis_index("s") * _NC + lax.axis_index("c")
        cbase = wid * _CHUNKS_PER_W
        pltpu.sync_copy(x_hbm.at[pl.ds(cbase, _CHUNKS_PER_W)], idx_v)
        pltpu.sync_copy(pos_hbm.at[pl.ds(0, _S)], pos_v.at[pl.ds(0, _S)])
        pltpu.sync_copy(pos_hbm.at[pl.ds(0, _S)], pos_v.at[pl.ds(_S, _S)])

        def start_gather(j, b):
            pltpu.make_async_copy(
                tok_hbm.at[idx_v.at[j]], inb.at[b], gsem.at[b]).start()

        def wait_gather(b):
            pltpu.make_async_copy(
                tok_hbm.at[idx_v.at[0]], inb.at[b], gsem.at[b]).wait()

        def start_scatter(j, b):
            pltpu.make_async_copy(
                outb.at[b],
                out_hbm.at[pl.ds((cbase + j) * _CHUNK, _CHUNK)],
                ssem.at[b]).start()

        def wait_scatter(b):
            pltpu.make_async_copy(
                outb.at[b],
                out_hbm.at[pl.ds(cbase * _CHUNK, _CHUNK)],
                ssem.at[b]).wait()

        def compute(j, b):
            base = lax.rem(j * _CHUNK, _S)

            def row_body(i, c2):
                s = base + i
                for d in range(_EMB // 16):
                    sl = pl.ds(d * 16, 16)
                    outb[b, i, sl] = inb[b, i, sl] * _SCALE + pos_v[s, sl]
                return c2

            lax.fori_loop(0, _CHUNK, row_body, 0)

        start_gather(0, 0)
        start_gather(1, 1)
        for j in (0, 1):
            wait_gather(j)
            compute(j, j)
            start_scatter(j, j)
            start_gather(j + 2, j)

        def pair_body(jj, carry):
            j = 2 * jj
            for b in (0, 1):
                wait_gather(b)
                wait_scatter(b)
                compute(j + b, b)
                start_scatter(j + b, b)
                start_gather(j + b + 2, b)
            return carry

        lax.fori_loop(1, _CHUNKS_PER_W // 2 - 1, pair_body, 0)

        for b in (0, 1):
            j = _CHUNKS_PER_W - 2 + b
            wait_gather(b)
            wait_scatter(b)
            compute(j, b)
            start_scatter(j, b)
        for b in (0, 1):
            wait_scatter(b)

    return k


_kernel_call = _make_kernel()


def kernel(x, token_table, pos_table):
    xf = x.reshape(_N // _CHUNK, _CHUNK).astype(jnp.int32)
    out = _kernel_call(xf, token_table, pos_table)
    return out.reshape(_B, _S, _EMB)


# speedup vs baseline: 1.4721x; 1.2527x over previous
import functools
import math

import jax
import jax.numpy as jnp
from jax import lax
from jax.experimental import pallas as pl
from jax.experimental.pallas import tpu as pltpu
from jax.experimental.pallas import tpu_sc as plsc

_VOCAB = 1000000
_EMB = 64
_S = 200
_B = 4096
_H = 100
_NC = 2
_NS = 16
_NW = _NC * _NS
_BPW = _B // _NW
_SCALE = math.sqrt(_EMB)


def _make_kernel():
    mesh = plsc.VectorSubcoreMesh(core_axis_name="c", subcore_axis_name="s")

    @functools.partial(
        pl.kernel,
        mesh=mesh,
        out_type=jax.ShapeDtypeStruct((_B, _S, _EMB), jnp.float32),
        compiler_params=pltpu.CompilerParams(use_tc_tiling_on_sc=False),
        scratch_types=[
            pltpu.VMEM((_BPW, 2, _H), jnp.int32),
            pltpu.VMEM((_S, _EMB), jnp.float32),
            pltpu.VMEM((2, _H, _EMB), jnp.float32),
            pltpu.VMEM((2, _H, _EMB), jnp.float32),
            pltpu.SemaphoreType.DMA((2,)),
            pltpu.SemaphoreType.DMA((2,)),
        ],
    )
    def k(x_hbm, tok_hbm, pos_hbm, out_hbm, idx_v, pos_v, inb, outb, gsem,
          ssem):
        wid = lax.axis_index("s") * _NC + lax.axis_index("c")
        bbase = wid * _BPW
        pltpu.sync_copy(x_hbm.at[pl.ds(bbase, _BPW)], idx_v)
        pltpu.sync_copy(pos_hbm.at[pl.ds(0, _S)], pos_v)

        def start_gather(r, h):
            pltpu.make_async_copy(
                tok_hbm.at[idx_v.at[r, h]], inb.at[h], gsem.at[h]).start()

        def wait_gather(h):
            pltpu.make_async_copy(
                tok_hbm.at[idx_v.at[0, h]], inb.at[h], gsem.at[h]).wait()

        def start_scatter(r, h):
            pltpu.make_async_copy(
                outb.at[h],
                out_hbm.at[bbase + r, pl.ds(h * _H, _H)],
                ssem.at[h]).start()

        def wait_scatter(h):
            pltpu.make_async_copy(
                outb.at[h],
                out_hbm.at[bbase, pl.ds(h * _H, _H)],
                ssem.at[h]).wait()

        def compute(h):
            base = h * _H

            def row_body(i, c2):
                for d in range(_EMB // 16):
                    sl = pl.ds(d * 16, 16)
                    outb[h, i, sl] = inb[h, i, sl] * _SCALE + pos_v[base + i,
                                                                    sl]
                return c2

            lax.fori_loop(0, _H, row_body, 0)

        start_gather(0, 0)
        start_gather(0, 1)
        for h in (0, 1):
            wait_gather(h)
            compute(h)
            start_scatter(0, h)
            start_gather(1, h)

        def row_loop(r, carry):
            for h in (0, 1):
                wait_gather(h)
                wait_scatter(h)
                compute(h)
                start_scatter(r, h)
                start_gather(r + 1, h)
            return carry

        lax.fori_loop(1, _BPW - 1, row_loop, 0)

        for h in (0, 1):
            wait_gather(h)
            wait_scatter(h)
            compute(h)
            start_scatter(_BPW - 1, h)
        for h in (0, 1):
            wait_scatter(h)

    return k


_kernel_call = _make_kernel()


def kernel(x, token_table, pos_table):
    x3 = x.astype(jnp.int32).reshape(_B, 2, _H)
    return _kernel_call(x3, token_table, pos_table)
